# baseline (speedup 1.0000x reference)
import functools

import jax
import jax.numpy as jnp
from jax import lax
from jax.experimental import pallas as pl
from jax.experimental.pallas import tpu as pltpu
from jax.experimental.pallas import tpu_sc as plsc

_ACTION_DIM = 7
_BATCH = 16384
_LANES = 16
_NC = 2
_NS = 16
_NW = _NC * _NS
_RPW = _BATCH // _NW


def _sanitize(x):
    return jnp.where(jnp.abs(x) < jnp.inf, x, jnp.float32(0.0))


def _mask_sc_kernel(cols_hbm, out_hbm, p0_vm, p4_vm, ex_vm, obuf, mbuf):
    wid = lax.axis_index("c") * _NS + lax.axis_index("s")
    base = wid * _RPW

    pltpu.sync_copy(cols_hbm.at[pl.ds(0, _BATCH)], p0_vm)
    pltpu.sync_copy(cols_hbm.at[pl.ds(_BATCH + base, _RPW)], p4_vm)
    pltpu.sync_copy(cols_hbm.at[pl.ds(2 * _BATCH + base, _RPW)], ex_vm)

    def _fold(i, m):
        return jnp.minimum(m, _sanitize(p0_vm[pl.ds(i * _LANES, _LANES)]))

    m = lax.fori_loop(0, _BATCH // _LANES, _fold,
                      jnp.full((_LANES,), jnp.inf, dtype=jnp.float32))
    mbuf[pl.ds(0, _LANES)] = m
    mbuf[pl.ds(_LANES, _LANES)] = m
    for s in (8, 4, 2, 1):
        v = jnp.minimum(mbuf[pl.ds(0, _LANES)], mbuf[pl.ds(s, _LANES)])
        mbuf[pl.ds(0, _LANES)] = v
        mbuf[pl.ds(_LANES, _LANES)] = v
    ones = jnp.ones((_LANES,), dtype=jnp.int32)
    zeros = jnp.zeros((_LANES,), dtype=jnp.int32)

    allhas32 = jnp.where(mbuf[pl.ds(0, _LANES)] > 0.5, ones, zeros)

    for j in range(_RPW // _LANES):
        o = j * _LANES
        p0 = _sanitize(p0_vm[pl.ds(base + o, _LANES)])
        p4 = _sanitize(p4_vm[pl.ds(o, _LANES)])
        ex = _sanitize(ex_vm[pl.ds(o, _LANES)])

        has32 = jnp.where(p0 > 0.5, ones, zeros)
        nothx32 = jnp.where(ex < 0.9, ones, zeros)
        asl32 = jnp.where(p4 >= 0.9, ones, zeros)

        buy32 = (ones - has32) * nothx32
        c632 = has32 * nothx32 * (ones - asl32 * allhas32)

        obuf[0, pl.ds(o, _LANES)] = ones
        obuf[1, pl.ds(o, _LANES)] = buy32
        obuf[2, pl.ds(o, _LANES)] = buy32
        obuf[3, pl.ds(o, _LANES)] = buy32
        obuf[4, pl.ds(o, _LANES)] = has32
        obuf[5, pl.ds(o, _LANES)] = has32
        obuf[6, pl.ds(o, _LANES)] = c632

    for c in range(_ACTION_DIM):
        pltpu.sync_copy(obuf.at[pl.ds(c, 1)],
                        out_hbm.at[pl.ds(c, 1), pl.ds(base, _RPW)])


@jax.jit
def kernel(position, portfolio):
    position = position.astype(jnp.float32)
    portfolio = portfolio.astype(jnp.float32)
    cols = jnp.concatenate(
        [position[:, 0], position[:, 4], portfolio[:, 2]], axis=0
    )
    mesh = plsc.VectorSubcoreMesh(core_axis_name="c", subcore_axis_name="s")
    sc_call = functools.partial(
        pl.kernel,
        mesh=mesh,
        out_type=jax.ShapeDtypeStruct((_ACTION_DIM, _BATCH), jnp.int32),
        scratch_types=[
            pltpu.VMEM((_BATCH,), jnp.float32),
            pltpu.VMEM((_RPW,), jnp.float32),
            pltpu.VMEM((_RPW,), jnp.float32),
            pltpu.VMEM((_ACTION_DIM, _RPW), jnp.int32),
            pltpu.VMEM((2 * _LANES,), jnp.float32),
        ],
    )(_mask_sc_kernel)
    raw = sc_call(cols)
    return raw.T.astype(jnp.bool_)


# speedup vs baseline: 1.2074x; 1.2074x over previous
import functools

import jax
import jax.numpy as jnp
from jax import lax
from jax.experimental import pallas as pl
from jax.experimental.pallas import tpu as pltpu
from jax.experimental.pallas import tpu_sc as plsc

_ACTION_DIM = 7
_BATCH = 16384
_LANES = 16
_NC = 2
_NS = 16
_NW = _NC * _NS
_RPW = _BATCH // _NW


def _sanitize(x):
    return jnp.where(jnp.abs(x) < jnp.inf, x, jnp.float32(0.0))


def _mask_sc_kernel(cols_hbm, out_hbm, p0_vm, p4_vm, ex_vm, obuf, mbuf,
                    red_vm, shared, parts_vm):
    cid = lax.axis_index("c")
    sid = lax.axis_index("s")
    wid = cid * _NS + sid
    base = wid * _RPW
    red_rows = _BATCH // _NS
    red_base = sid * red_rows

    pltpu.sync_copy(cols_hbm.at[pl.ds(red_base, red_rows)], red_vm)
    pltpu.sync_copy(cols_hbm.at[pl.ds(base, _RPW)], p0_vm)
    pltpu.sync_copy(cols_hbm.at[pl.ds(_BATCH + base, _RPW)], p4_vm)
    pltpu.sync_copy(cols_hbm.at[pl.ds(2 * _BATCH + base, _RPW)], ex_vm)

    m = _sanitize(red_vm[pl.ds(0, _LANES)])
    for i in range(1, red_rows // _LANES):
        m = jnp.minimum(m, _sanitize(red_vm[pl.ds(i * _LANES, _LANES)]))

    mbuf[pl.ds(0, _LANES)] = m
    pltpu.sync_copy(mbuf.at[pl.ds(0, _LANES)], shared.at[sid])
    plsc.subcore_barrier()
    pltpu.sync_copy(shared, parts_vm)
    m = parts_vm[0, pl.ds(0, _LANES)]
    for i in range(1, _NS):
        m = jnp.minimum(m, parts_vm[i, pl.ds(0, _LANES)])

    mbuf[pl.ds(0, _LANES)] = m
    mbuf[pl.ds(_LANES, _LANES)] = m
    for s in (8, 4, 2, 1):
        v = jnp.minimum(mbuf[pl.ds(0, _LANES)], mbuf[pl.ds(s, _LANES)])
        mbuf[pl.ds(0, _LANES)] = v
        mbuf[pl.ds(_LANES, _LANES)] = v
    ones = jnp.ones((_LANES,), dtype=jnp.int32)
    zeros = jnp.zeros((_LANES,), dtype=jnp.int32)

    allhas32 = jnp.where(mbuf[pl.ds(0, _LANES)] > 0.5, ones, zeros)

    for j in range(_RPW // _LANES):
        o = j * _LANES
        p0 = _sanitize(p0_vm[pl.ds(o, _LANES)])
        p4 = _sanitize(p4_vm[pl.ds(o, _LANES)])
        ex = _sanitize(ex_vm[pl.ds(o, _LANES)])

        has32 = jnp.where(p0 > 0.5, ones, zeros)
        nothx32 = jnp.where(ex < 0.9, ones, zeros)
        asl32 = jnp.where(p4 >= 0.9, ones, zeros)

        buy32 = (ones - has32) * nothx32
        c632 = has32 * nothx32 * (ones - asl32 * allhas32)

        obuf[0, pl.ds(o, _LANES)] = ones
        obuf[1, pl.ds(o, _LANES)] = buy32
        obuf[2, pl.ds(o, _LANES)] = buy32
        obuf[3, pl.ds(o, _LANES)] = buy32
        obuf[4, pl.ds(o, _LANES)] = has32
        obuf[5, pl.ds(o, _LANES)] = has32
        obuf[6, pl.ds(o, _LANES)] = c632

    for c in range(_ACTION_DIM):
        pltpu.sync_copy(obuf.at[pl.ds(c, 1)],
                        out_hbm.at[pl.ds(c, 1), pl.ds(base, _RPW)])


@jax.jit
def kernel(position, portfolio):
    position = position.astype(jnp.float32)
    portfolio = portfolio.astype(jnp.float32)
    cols = jnp.concatenate(
        [position[:, 0], position[:, 4], portfolio[:, 2]], axis=0
    )
    mesh = plsc.VectorSubcoreMesh(core_axis_name="c", subcore_axis_name="s")
    sc_call = functools.partial(
        pl.kernel,
        mesh=mesh,
        out_type=jax.ShapeDtypeStruct((_ACTION_DIM, _BATCH), jnp.int32),
        scratch_types=[
            pltpu.VMEM((_RPW,), jnp.float32),
            pltpu.VMEM((_RPW,), jnp.float32),
            pltpu.VMEM((_RPW,), jnp.float32),
            pltpu.VMEM((_ACTION_DIM, _RPW), jnp.int32),
            pltpu.VMEM((2 * _LANES,), jnp.float32),
            pltpu.VMEM((_BATCH // _NS,), jnp.float32),
            pltpu.VMEM_SHARED((_NS, _LANES), jnp.float32),
            pltpu.VMEM((_NS, _LANES), jnp.float32),
        ],
    )(_mask_sc_kernel)
    raw = sc_call(cols)
    return raw.T.astype(jnp.bool_)


# speedup vs baseline: 3.8182x; 3.1625x over previous
import jax
import jax.numpy as jnp
from jax.experimental import pallas as pl

_ACTION_DIM = 7


def _sanitize(x):
    return jnp.where(jnp.isfinite(x), x, 0.0)


def _mask_kernel(p0_ref, p4_ref, ex_ref, out_ref):
    n = out_ref.shape[1]
    p0 = _sanitize(p0_ref[...])
    p4 = _sanitize(p4_ref[...])
    ex = _sanitize(ex_ref[...])

    has = p0 > 0.5
    hx = ex >= 0.9
    asl = p4 >= 0.9

    all_has = jnp.min(p0) > 0.5

    not_hx = jnp.logical_not(hx)
    buy = jnp.logical_not(has) & not_hx
    c6 = has & not_hx & jnp.logical_not(jnp.logical_and(all_has, asl))

    buy8 = buy.astype(jnp.int8).reshape(1, n)
    has8 = has.astype(jnp.int8).reshape(1, n)
    c68 = c6.astype(jnp.int8).reshape(1, n)

    out_ref[0:1, :] = jnp.ones((1, n), dtype=jnp.int8)
    out_ref[1:2, :] = buy8
    out_ref[2:3, :] = buy8
    out_ref[3:4, :] = buy8
    out_ref[4:5, :] = has8
    out_ref[5:6, :] = has8
    out_ref[6:7, :] = c68


@jax.jit
def kernel(position, portfolio):
    position = position.astype(jnp.float32)
    portfolio = portfolio.astype(jnp.float32)
    batch = position.shape[0]
    raw = pl.pallas_call(
        _mask_kernel,
        out_shape=jax.ShapeDtypeStruct((_ACTION_DIM, batch), jnp.int8),
    )(position[:, 0], position[:, 4], portfolio[:, 2])
    return raw.T.astype(jnp.bool_)


# speedup vs baseline: 4.1194x; 1.0789x over previous
import jax
import jax.numpy as jnp
from jax.experimental import pallas as pl

_ACTION_DIM = 7


def _sanitize(x):
    return jnp.where(jnp.isfinite(x), x, 0.0)


def _mask_kernel(cols_ref, out_ref):
    n = out_ref.shape[1]
    x = cols_ref[...]
    p0 = _sanitize(x[0:n])
    p4 = _sanitize(x[n:2 * n])
    ex = _sanitize(x[2 * n:3 * n])

    has = p0 > 0.5
    hx = ex >= 0.9
    asl = p4 >= 0.9

    all_has = jnp.min(p0) > 0.5

    not_hx = jnp.logical_not(hx)
    buy = jnp.logical_not(has) & not_hx
    c6 = has & not_hx & jnp.logical_not(jnp.logical_and(all_has, asl))

    buy8 = buy.astype(jnp.int8).reshape(1, n)
    has8 = has.astype(jnp.int8).reshape(1, n)
    c68 = c6.astype(jnp.int8).reshape(1, n)

    out_ref[0:1, :] = jnp.ones((1, n), dtype=jnp.int8)
    out_ref[1:2, :] = buy8
    out_ref[2:3, :] = buy8
    out_ref[3:4, :] = buy8
    out_ref[4:5, :] = has8
    out_ref[5:6, :] = has8
    out_ref[6:7, :] = c68


@jax.jit
def kernel(position, portfolio):
    position = position.astype(jnp.float32)
    portfolio = portfolio.astype(jnp.float32)
    batch = position.shape[0]
    cols = jnp.concatenate(
        [position[:, 0], position[:, 4], portfolio[:, 2]], axis=0
    )
    raw = pl.pallas_call(
        _mask_kernel,
        out_shape=jax.ShapeDtypeStruct((_ACTION_DIM, batch), jnp.int8),
    )(cols)
    return raw.T.astype(jnp.bool_)
